# baseline (speedup 1.0000x reference)
import functools

import jax
import jax.numpy as jnp
from jax import lax
from jax.experimental import pallas as pl
from jax.experimental.pallas import tpu as pltpu
from jax.experimental.pallas import tpu_sc as plsc

B = 16384
D = 64
NUM_WORKERS = 32
BPW = B // NUM_WORKERS
CHUNK = 256
NCHUNK = BPW // CHUNK
CROWS = 2000


def _compact_body(a_ref, b_ref, o_ref):
    o_ref[...] = jnp.concatenate([a_ref[...], b_ref[...]], axis=1)


def _compact(tab):
    n = tab.shape[0]
    nb = (n // 2) // CROWS
    return pl.pallas_call(
        _compact_body,
        grid=(nb,),
        in_specs=[
            pl.BlockSpec((CROWS, D), lambda i: (i, 0)),
            pl.BlockSpec((CROWS, D), lambda i, nb=nb: (i + nb, 0)),
        ],
        out_specs=pl.BlockSpec((CROWS, 2 * D), lambda i: (i, 0)),
        out_shape=jax.ShapeDtypeStruct((n // 2, 2 * D), jnp.float32),
    )(tab, tab)


def _body(pu_hbm, su_hbm, pp_hbm, sp_hbm, utab_hbm, ptab_hbm, out_hbm,
          idx_u, off_u, idx_p, off_p, urows, prows, outv, sem_u, sem_p):
    wid = lax.axis_index("s") * 2 + lax.axis_index("c")
    base = wid * BPW

    pltpu.sync_copy(pu_hbm.at[pl.ds(base, BPW)], idx_u)
    pltpu.sync_copy(su_hbm.at[pl.ds(base, BPW)], off_u)
    pltpu.sync_copy(pp_hbm.at[pl.ds(base, BPW)], idx_p)
    pltpu.sync_copy(sp_hbm.at[pl.ds(base, BPW)], off_p)

    lanes = lax.iota(jnp.int32, 16)

    def c_body(c, carry):
        cu = pltpu.async_copy(
            utab_hbm.at[idx_u.at[pl.ds(c * CHUNK, CHUNK)]], urows, sem_u)
        cp = pltpu.async_copy(
            ptab_hbm.at[idx_p.at[pl.ds(c * CHUNK, CHUNK)]], prows, sem_p)
        cu.wait()
        cp.wait()

        for g in range(CHUNK // 16):
            local = g * 16 + lanes
            su = off_u[pl.ds(c * CHUNK + g * 16, 16)]
            sp = off_p[pl.ds(c * CHUNK + g * 16, 16)]

            def d_body(d, acc):
                cols_u = su + d
                cols_p = sp + d
                u = plsc.load_gather(urows, [local, cols_u])
                p = plsc.load_gather(prows, [local, cols_p])
                return acc + u * p

            acc = lax.fori_loop(0, D, d_body, jnp.zeros((16,), jnp.float32),
                                unroll=8)
            outv[pl.ds(c * CHUNK + g * 16, 16)] = acc
        return carry

    lax.fori_loop(0, NCHUNK, c_body, 0)
    pltpu.sync_copy(outv, out_hbm.at[pl.ds(base, BPW)])


@jax.jit
def _run(pu, su, pp, sp, user_table, place_table):
    ut2 = _compact(user_table)
    pt2 = _compact(place_table)
    mesh = plsc.VectorSubcoreMesh(core_axis_name="c", subcore_axis_name="s")
    k = functools.partial(
        pl.kernel,
        mesh=mesh,
        compiler_params=pltpu.CompilerParams(
            needs_layout_passes=False, use_tc_tiling_on_sc=True),
        out_type=jax.ShapeDtypeStruct((B,), jnp.float32),
        scratch_types=[
            pltpu.VMEM((BPW,), jnp.int32),
            pltpu.VMEM((BPW,), jnp.int32),
            pltpu.VMEM((BPW,), jnp.int32),
            pltpu.VMEM((BPW,), jnp.int32),
            pltpu.VMEM((CHUNK, 2 * D), jnp.float32),
            pltpu.VMEM((CHUNK, 2 * D), jnp.float32),
            pltpu.VMEM((BPW,), jnp.float32),
            pltpu.SemaphoreType.DMA,
            pltpu.SemaphoreType.DMA,
        ],
    )(_body)
    return k(pu, su, pp, sp, ut2, pt2)


def kernel(inputs, user_table, place_table):
    uids = inputs[:, 0].astype(jnp.int32)
    pids = inputs[:, 1].astype(jnp.int32)
    half = user_table.shape[0] // 2
    out = _run(uids % half, (uids // half) * D, pids % half,
               (pids // half) * D, user_table, place_table)
    return out.reshape(B, 1)


# speedup vs baseline: 1.1845x; 1.1845x over previous
import functools

import jax
import jax.numpy as jnp
from jax import lax
from jax.experimental import pallas as pl
from jax.experimental.pallas import tpu as pltpu
from jax.experimental.pallas import tpu_sc as plsc

B = 16384
D = 64
NUM_WORKERS = 32
BPW = B // NUM_WORKERS
CHUNK = 256
NCHUNK = BPW // CHUNK
CROWS = 2000


def _compact(tab):
    n = tab.shape[0]
    return tab.reshape(n // 2, 2 * D)


def _body(pu_hbm, su_hbm, pp_hbm, sp_hbm, utab_hbm, ptab_hbm, out_hbm,
          idx_u, off_u, idx_p, off_p, urows, prows, outv, sem_u, sem_p):
    wid = lax.axis_index("s") * 2 + lax.axis_index("c")
    base = wid * BPW

    pltpu.sync_copy(pu_hbm.at[pl.ds(base, BPW)], idx_u)
    pltpu.sync_copy(su_hbm.at[pl.ds(base, BPW)], off_u)
    pltpu.sync_copy(pp_hbm.at[pl.ds(base, BPW)], idx_p)
    pltpu.sync_copy(sp_hbm.at[pl.ds(base, BPW)], off_p)

    lanes = lax.iota(jnp.int32, 16)

    def c_body(c, carry):
        cu = pltpu.async_copy(
            utab_hbm.at[idx_u.at[pl.ds(c * CHUNK, CHUNK)]], urows, sem_u)
        cp = pltpu.async_copy(
            ptab_hbm.at[idx_p.at[pl.ds(c * CHUNK, CHUNK)]], prows, sem_p)
        cu.wait()
        cp.wait()

        for g in range(CHUNK // 16):
            local = g * 16 + lanes
            su = off_u[pl.ds(c * CHUNK + g * 16, 16)]
            sp = off_p[pl.ds(c * CHUNK + g * 16, 16)]

            def d_body(d, acc):
                cols_u = su + d
                cols_p = sp + d
                u = plsc.load_gather(urows, [local, cols_u])
                p = plsc.load_gather(prows, [local, cols_p])
                return acc + u * p

            acc = lax.fori_loop(0, D, d_body, jnp.zeros((16,), jnp.float32),
                                unroll=8)
            outv[pl.ds(c * CHUNK + g * 16, 16)] = acc
        return carry

    lax.fori_loop(0, NCHUNK, c_body, 0)
    pltpu.sync_copy(outv, out_hbm.at[pl.ds(base, BPW)])


@jax.jit
def _run(pu, su, pp, sp, user_table, place_table):
    ut2 = _compact(user_table)
    pt2 = _compact(place_table)
    mesh = plsc.VectorSubcoreMesh(core_axis_name="c", subcore_axis_name="s")
    k = functools.partial(
        pl.kernel,
        mesh=mesh,
        compiler_params=pltpu.CompilerParams(
            needs_layout_passes=False, use_tc_tiling_on_sc=True),
        out_type=jax.ShapeDtypeStruct((B,), jnp.float32),
        scratch_types=[
            pltpu.VMEM((BPW,), jnp.int32),
            pltpu.VMEM((BPW,), jnp.int32),
            pltpu.VMEM((BPW,), jnp.int32),
            pltpu.VMEM((BPW,), jnp.int32),
            pltpu.VMEM((CHUNK, 2 * D), jnp.float32),
            pltpu.VMEM((CHUNK, 2 * D), jnp.float32),
            pltpu.VMEM((BPW,), jnp.float32),
            pltpu.SemaphoreType.DMA,
            pltpu.SemaphoreType.DMA,
        ],
    )(_body)
    return k(pu, su, pp, sp, ut2, pt2)


def kernel(inputs, user_table, place_table):
    uids = inputs[:, 0].astype(jnp.int32)
    pids = inputs[:, 1].astype(jnp.int32)
    out = _run(uids >> 1, (uids & 1) * D, pids >> 1, (pids & 1) * D,
               user_table, place_table)
    return out.reshape(B, 1)


# speedup vs baseline: 1.3017x; 1.0989x over previous
import functools

import jax
import jax.numpy as jnp
from jax import lax
from jax.experimental import pallas as pl
from jax.experimental.pallas import tpu as pltpu
from jax.experimental.pallas import tpu_sc as plsc

B = 16384
D = 64
NROWS = 1000000
NUM_WORKERS = 32
BPW = B // NUM_WORKERS
WIN1 = 512
WIN2 = 256


def _wid():
    return lax.axis_index("s") * 2 + lax.axis_index("c")


def _scal(vec, lane):
    idx = (jnp.zeros((16,), jnp.int32) + lane)[:, None]
    dnums = lax.GatherDimensionNumbers(
        offset_dims=(), collapsed_slice_dims=(0,), start_index_map=(0,))
    picked = lax.gather(vec, idx, dnums, (1,),
                        mode=lax.GatherScatterMode.PROMISE_IN_BOUNDS)
    return picked[0]


def _sweep1_body(uid_hbm, utab_hbm, uinter_hbm, idx_v, winbuf, asm, sem,
                 wsem):
    w = _wid()
    base = w * BPW
    pltpu.sync_copy(uid_hbm.at[pl.ds(base, BPW)], idx_v)
    lanes = lax.iota(jnp.int32, 16)

    def outer_cond(c):
        return c[0] < BPW

    def outer(c):
        s0, _ = c
        cur = idx_v[pl.ds((s0 >> 4) << 4, 16)]
        first = _scal(cur, s0 & 15)
        t = jnp.minimum((first >> 3) << 3, NROWS - WIN1)
        t = pl.multiple_of(t, 8)
        pltpu.async_copy(utab_hbm.at[pl.ds(t, WIN1)], winbuf, wsem).wait()

        def inner_cond(c2):
            s, _ = c2
            vec = idx_v[pl.ds((s >> 4) << 4, 16)]
            row = _scal(vec, s & 15)
            return jnp.logical_and(s < BPW, row < t + WIN1)

        def inner(c2):
            s, carry = c2
            vec = idx_v[pl.ds((s >> 4) << 4, 16)]
            rl = _scal(vec, s & 15) - t
            rows = jnp.zeros((16,), jnp.int32) + rl
            arow = jnp.zeros((16,), jnp.int32) + (s >> 1)
            cbase = (s & 1) * D
            for cch in range(D // 16):
                v = plsc.load_gather(winbuf, [rows, cch * 16 + lanes])
                plsc.store_scatter(asm, [arow, cbase + cch * 16 + lanes], v)
            return (s + 1, carry)

        s1, _ = lax.while_loop(inner_cond, inner, (s0, 0))
        return (s1, 0)

    lax.while_loop(outer_cond, outer, (0, 0))
    pltpu.sync_copy(asm, uinter_hbm.at[pl.ds(w * (BPW // 2), BPW // 2)])


def _sweep2_body(pid_hbm, ru_hbm, ptab_hbm, uinter_hbm, out_hbm,
                 idx_v, ru_v, pairidx, ubuf, winbuf, outv, sem, wsem):
    w = _wid()
    base = w * BPW
    pltpu.sync_copy(pid_hbm.at[pl.ds(base, BPW)], idx_v)
    pltpu.sync_copy(ru_hbm.at[pl.ds(base, BPW)], ru_v)
    lanes = lax.iota(jnp.int32, 16)

    def mk_pair(g, carry):
        pairidx[pl.ds(g * 16, 16)] = ru_v[pl.ds(g * 16, 16)] >> 1
        return carry

    lax.fori_loop(0, BPW // 16, mk_pair, 0)
    pltpu.async_copy(uinter_hbm.at[pairidx], ubuf, sem).wait()

    def outer_cond(c):
        return c[0] < BPW

    def outer(c):
        s0, gacc = c
        cur = idx_v[pl.ds((s0 >> 4) << 4, 16)]
        first = _scal(cur, s0 & 15)
        t = jnp.minimum((first >> 3) << 3, NROWS - WIN2)
        t = pl.multiple_of(t, 8)
        pltpu.async_copy(ptab_hbm.at[pl.ds(t, WIN2)], winbuf, wsem).wait()

        def inner_cond(c2):
            s, _ = c2
            vec = idx_v[pl.ds((s >> 4) << 4, 16)]
            row = _scal(vec, s & 15)
            return jnp.logical_and(s < BPW, row < t + WIN2)

        def inner(c2):
            s, acc = c2
            vec = idx_v[pl.ds((s >> 4) << 4, 16)]
            rl = _scal(vec, s & 15) - t
            ruvec = ru_v[pl.ds((s >> 4) << 4, 16)]
            ru = _scal(ruvec, s & 15)
            prow = jnp.zeros((16,), jnp.int32) + rl
            urow = jnp.zeros((16,), jnp.int32) + s
            ub = (ru & 1) * D
            tot = jnp.zeros((16,), jnp.float32)
            for cch in range(D // 16):
                pv = plsc.load_gather(winbuf, [prow, cch * 16 + lanes])
                uv = plsc.load_gather(ubuf, [urow, ub + cch * 16 + lanes])
                tot = tot + pv * uv
            dot = jnp.sum(tot)
            acc = jnp.where(lanes == (s & 15), dot, acc)

            @pl.when((s & 15) == 15)
            def _():
                outv[pl.ds((s >> 4) << 4, 16)] = acc

            return (s + 1, acc)

        s1, gacc1 = lax.while_loop(inner_cond, inner, (s0, gacc))
        return (s1, gacc1)

    lax.while_loop(outer_cond, outer, (0, jnp.zeros((16,), jnp.float32)))
    pltpu.sync_copy(outv, out_hbm.at[pl.ds(base, BPW)])


@jax.jit
def _run(uid_s, pid_s, ru_s, user_table, place_table):
    mesh = plsc.VectorSubcoreMesh(core_axis_name="c", subcore_axis_name="s")
    params = pltpu.CompilerParams(
        needs_layout_passes=False, use_tc_tiling_on_sc=True)

    sweep1 = functools.partial(
        pl.kernel,
        mesh=mesh,
        compiler_params=params,
        out_type=jax.ShapeDtypeStruct((B // 2, 2 * D), jnp.float32),
        scratch_types=[
            pltpu.VMEM((BPW,), jnp.int32),
            pltpu.VMEM((WIN1, D), jnp.float32),
            pltpu.VMEM((BPW // 2, 2 * D), jnp.float32),
            pltpu.SemaphoreType.DMA,
            pltpu.SemaphoreType.DMA,
        ],
    )(_sweep1_body)
    uinter = sweep1(uid_s, user_table)

    sweep2 = functools.partial(
        pl.kernel,
        mesh=mesh,
        compiler_params=params,
        out_type=jax.ShapeDtypeStruct((B,), jnp.float32),
        scratch_types=[
            pltpu.VMEM((BPW,), jnp.int32),
            pltpu.VMEM((BPW,), jnp.int32),
            pltpu.VMEM((BPW,), jnp.int32),
            pltpu.VMEM((BPW, 2 * D), jnp.float32),
            pltpu.VMEM((WIN2, D), jnp.float32),
            pltpu.VMEM((BPW,), jnp.float32),
            pltpu.SemaphoreType.DMA,
            pltpu.SemaphoreType.DMA,
        ],
    )(_sweep2_body)
    return sweep2(pid_s, ru_s, place_table, uinter)


def kernel(inputs, user_table, place_table):
    uids = inputs[:, 0].astype(jnp.int32)
    pids = inputs[:, 1].astype(jnp.int32)
    perm_u = jnp.argsort(uids)
    perm_p = jnp.argsort(pids)
    inv_u = jnp.argsort(perm_u)
    uid_s = uids[perm_u]
    pid_s = pids[perm_p]
    ru_s = inv_u[perm_p].astype(jnp.int32)
    out_sorted = _run(uid_s, pid_s, ru_s, user_table, place_table)
    out = out_sorted[jnp.argsort(perm_p)]
    return out.reshape(B, 1)


# speedup vs baseline: 1.8479x; 1.4196x over previous
import functools

import jax
import jax.numpy as jnp
from jax import lax
from jax.experimental import pallas as pl
from jax.experimental.pallas import tpu as pltpu
from jax.experimental.pallas import tpu_sc as plsc

B = 16384
D = 64
NUM_WORKERS = 32
BPW = B // NUM_WORKERS
CHUNK = 256
NCHUNK = BPW // CHUNK


def _body(uids_hbm, pids_hbm, utab_hbm, ptab_hbm, out_hbm,
          idx_u, idx_p, urows, prows, outv, sem_u, sem_p, sem_u2, sem_p2):
    wid = lax.axis_index("s") * 2 + lax.axis_index("c")
    base = wid * BPW

    pltpu.sync_copy(uids_hbm.at[pl.ds(base, BPW)], idx_u)
    pltpu.sync_copy(pids_hbm.at[pl.ds(base, BPW)], idx_p)

    lanes = lax.iota(jnp.int32, 16)

    def c_body(c, carry):
        def f_body(g, fc):
            iu = idx_u[pl.ds(c * CHUNK + g * 16, 16)]
            ip = idx_p[pl.ds(c * CHUNK + g * 16, 16)]
            for j in range(16):
                jj = g * 16 + j
                su = sem_u if j % 2 == 0 else sem_u2
                sp = sem_p if j % 2 == 0 else sem_p2
                pltpu.async_copy(utab_hbm.at[iu[j]], urows.at[jj], su)
                pltpu.async_copy(ptab_hbm.at[ip[j]], prows.at[jj], sp)
            return fc

        lax.fori_loop(0, CHUNK // 16, f_body, 0)

        half = CHUNK // 2
        pltpu.make_async_copy(utab_hbm.at[pl.ds(0, half)],
                              urows.at[pl.ds(0, half)], sem_u).wait()
        pltpu.make_async_copy(utab_hbm.at[pl.ds(0, half)],
                              urows.at[pl.ds(half, half)], sem_u2).wait()
        pltpu.make_async_copy(ptab_hbm.at[pl.ds(0, half)],
                              prows.at[pl.ds(0, half)], sem_p).wait()
        pltpu.make_async_copy(ptab_hbm.at[pl.ds(0, half)],
                              prows.at[pl.ds(half, half)], sem_p2).wait()

        for g in range(CHUNK // 16):
            local = g * 16 + lanes

            def d_body(d, acc):
                cols = jnp.zeros((16,), jnp.int32) + d
                u = plsc.load_gather(urows, [local, cols])
                p = plsc.load_gather(prows, [local, cols])
                return acc + u * p

            acc = lax.fori_loop(0, D, d_body, jnp.zeros((16,), jnp.float32),
                                unroll=8)
            outv[pl.ds(c * CHUNK + g * 16, 16)] = acc
        return carry

    lax.fori_loop(0, NCHUNK, c_body, 0)
    pltpu.sync_copy(outv, out_hbm.at[pl.ds(base, BPW)])


@jax.jit
def _run(uids, pids, user_table, place_table):
    mesh = plsc.VectorSubcoreMesh(core_axis_name="c", subcore_axis_name="s")
    k = functools.partial(
        pl.kernel,
        mesh=mesh,
        compiler_params=pltpu.CompilerParams(
            needs_layout_passes=False, use_tc_tiling_on_sc=True),
        out_type=jax.ShapeDtypeStruct((B,), jnp.float32),
        scratch_types=[
            pltpu.VMEM((BPW,), jnp.int32),
            pltpu.VMEM((BPW,), jnp.int32),
            pltpu.VMEM((CHUNK, D), jnp.float32),
            pltpu.VMEM((CHUNK, D), jnp.float32),
            pltpu.VMEM((BPW,), jnp.float32),
            pltpu.SemaphoreType.DMA,
            pltpu.SemaphoreType.DMA,
            pltpu.SemaphoreType.DMA,
            pltpu.SemaphoreType.DMA,
        ],
    )(_body)
    return k(uids, pids, user_table, place_table)


def kernel(inputs, user_table, place_table):
    uids = inputs[:, 0].astype(jnp.int32)
    pids = inputs[:, 1].astype(jnp.int32)
    out = _run(uids, pids, user_table, place_table)
    return out.reshape(B, 1)
